# baseline (speedup 1.0000x reference)
import jax
import jax.numpy as jnp
from jax.experimental import pallas as pl
from jax.experimental.pallas import tpu as pltpu

_B, _H, _S, _D = 16, 16, 4096, 64
_L = 16
_SB = 2048


def _body(pos_ref, kc, vc, kv, vv, ko, vo):
    j = pl.program_id(1)
    ko[...] = kc[...]
    vo[...] = vc[...]
    base = j * _SB
    for l in range(_L):
        p = pos_ref[l] - base

        @pl.when((p >= 0) & (p < _SB))
        def _():
            ko[0, pl.ds(p, 1), :] = kv[0, pl.ds(l, 1), :]
            vo[0, pl.ds(p, 1), :] = vv[0, pl.ds(l, 1), :]


def kernel(k_cache, v_cache, input_pos, k_val, v_val):
    kc = k_cache.reshape(_B * _H, _S, _D)
    vc = v_cache.reshape(_B * _H, _S, _D)
    kvl = k_val.reshape(_B * _H, _L, _D)
    vvl = v_val.reshape(_B * _H, _L, _D)
    grid = (_B * _H, _S // _SB)
    cache_spec = pl.BlockSpec((1, _SB, _D), lambda i, j, pos: (i, j, 0))
    val_spec = pl.BlockSpec((1, _L, _D), lambda i, j, pos: (i, 0, 0))
    ko, vo = pl.pallas_call(
        _body,
        grid_spec=pltpu.PrefetchScalarGridSpec(
            num_scalar_prefetch=1,
            grid=grid,
            in_specs=[cache_spec, cache_spec, val_spec, val_spec],
            out_specs=[cache_spec, cache_spec],
        ),
        out_shape=[jax.ShapeDtypeStruct((_B * _H, _S, _D), jnp.float32)] * 2,
        compiler_params=pltpu.CompilerParams(
            dimension_semantics=("parallel", "parallel"),
        ),
    )(input_pos, kc, vc, kvl, vvl)
    return ko.reshape(_B, _H, _S, _D), vo.reshape(_B, _H, _S, _D)
